# Initial kernel scaffold; baseline (speedup 1.0000x reference)
import jax
import jax.numpy as jnp
from jax.experimental import pallas as pl


def kernel(label, clsctx, token_prefix, token_suffix):
    raise NotImplementedError("write your pallas kernel here")


# speedup vs baseline: 1.0170x; 1.0170x over previous
import jax
import jax.numpy as jnp
from jax import lax
from jax.experimental import pallas as pl
from jax.experimental.pallas import tpu as pltpu
from jax.experimental.pallas import tpu_sc as plsc

_NUM_CLASS = 100000
_CTX = 4
_D = 512
_B = 4096
_PRE = 5
_SUF = 68
_SEQ = 77
_L = 16

_NC = 2
_NS = 16
_NW = _NC * _NS
_BPW = _B // _NW

_K = 16


def _body(lab_h, tab_h, static_h, prom_o, cls_o,
          lab_v, gbuf, slab0, slab1, gsem, sem0, sem1):
    c = lax.axis_index("c")
    s = lax.axis_index("s")
    wid = s * _NC + c
    base = wid * _BPW

    pltpu.sync_copy(lab_h.at[pl.ds(base, _BPW)], lab_v)
    pltpu.sync_copy(static_h, slab0)
    pltpu.sync_copy(static_h, slab1)

    def chunk(ci, carry):
        cb = base + ci * _K
        idx = lab_v[pl.ds(ci * _K, _K)]
        pltpu.async_copy(tab_h.at[idx], gbuf, gsem).wait()
        pltpu.sync_copy(gbuf, cls_o.at[pl.ds(cb, _K)])

        def pair(p, carry2):
            g = ci * (_K // 2) + p

            @pl.when(g > 0)
            def _():
                pltpu.make_async_copy(slab0, prom_o.at[0], sem0).wait()
                pltpu.make_async_copy(slab1, prom_o.at[0], sem1).wait()

            for sl, slab, sem in ((0, slab0, sem0), (1, slab1, sem1)):
                i = p * 2 + sl
                for j in range(_CTX):
                    for l in range(_D // _L):
                        slab[_PRE + j, pl.ds(l * _L, _L)] = (
                            gbuf[i, j, pl.ds(l * _L, _L)])
                pltpu.async_copy(slab, prom_o.at[cb + i], sem)
            return carry2

        lax.fori_loop(0, _K // 2, pair, 0)
        return carry

    lax.fori_loop(0, _BPW // _K, chunk, 0)
    pltpu.make_async_copy(slab0, prom_o.at[0], sem0).wait()
    pltpu.make_async_copy(slab1, prom_o.at[0], sem1).wait()


@jax.jit
def kernel(label, clsctx, token_prefix, token_suffix):
    label32 = label.astype(jnp.int32)
    pfx = token_prefix.reshape(_PRE, _D)
    sfx = token_suffix.reshape(_SUF, _D)
    static = jnp.concatenate(
        [pfx, jnp.zeros((_CTX, _D), jnp.float32), sfx], axis=0)

    call = pl.kernel(
        _body,
        out_type=(
            jax.ShapeDtypeStruct((_B, _SEQ, _D), jnp.float32),
            jax.ShapeDtypeStruct((_B, _CTX, _D), jnp.float32),
        ),
        mesh=plsc.VectorSubcoreMesh(core_axis_name="c", subcore_axis_name="s",
                                    num_cores=_NC, num_subcores=_NS),
        scratch_types=[
            pltpu.VMEM((_BPW,), jnp.int32),
            pltpu.VMEM((_K, _CTX, _D), jnp.float32),
            pltpu.VMEM((_SEQ, _D), jnp.float32),
            pltpu.VMEM((_SEQ, _D), jnp.float32),
            pltpu.SemaphoreType.DMA,
            pltpu.SemaphoreType.DMA,
            pltpu.SemaphoreType.DMA,
        ],
    )
    prompts, cls = call(label32, clsctx, static)
    return prompts, cls.reshape(_B, _CTX * _D)
